# baseline (speedup 1.0000x reference)
import functools

import jax
import jax.numpy as jnp
from jax import lax
from jax.experimental import pallas as pl
from jax.experimental.pallas import tpu as pltpu
from jax.experimental.pallas import tpu_sc as plsc

BATCH = 16384
D = 64
H1 = 128
V = 100000
FPS = 4
SEG = 33408
SEGS = [(0, SEG, SEG),
        (SEG, SEG, 2 * SEG),
        (2 * SEG, V - 32 - 2 * SEG, V)]
BSEG = 6400
NG = BATCH // 16


def _lookup_seg(ids_v, buf, out_v, base, hi, first):
    @plsc.parallel_loop(0, NG, step=1, unroll=8)
    def body(i):
        iv = ids_v[pl.ds(i * 16, 16)]
        m = (iv >= base) & (iv < hi) if base else iv < hi
        g = plsc.load_gather(buf, [iv - base if base else iv], mask=m)
        if first:
            out_v[pl.ds(i * 16, 16)] = jnp.where(m, g, 0.0)
        else:
            pos = lax.iota(jnp.int32, 16) + i * 16
            plsc.store_scatter(out_v, [pos], g, mask=m)


def _do_table(tbl_h, tail_h, bias_h, ids_h, emb_h, bias_out_h, s,
              ids_v, buf_a, buf_b, out_v, sem_a, sem_b):
    bufs = (buf_a, buf_b)
    sems = (sem_a, sem_b)
    pltpu.sync_copy(ids_h, ids_v)

    units = [(k, t) for k in range(FPS) for t in range(3)]

    def issue(u):
        k, t = units[u]
        return pltpu.async_copy(
            tbl_h.at[s * FPS + k, pl.ds(SEGS[t][0], SEGS[t][1])],
            bufs[u % 2].at[pl.ds(0, SEGS[t][1])], sems[u % 2])

    descs = {0: issue(0)}
    for u in range(len(units)):
        if u + 1 < len(units):
            descs[u + 1] = issue(u + 1)
        descs[u].wait()
        k, t = units[u]
        if t == 2:
            pltpu.sync_copy(
                tail_h.at[pl.ds(pl.multiple_of((s * FPS + k) * 128, 128),
                                128)],
                bufs[u % 2].at[pl.ds(SEGS[2][1], 128)])
        _lookup_seg(ids_v, bufs[u % 2], out_v, SEGS[t][0], SEGS[t][2],
                    t == 0)
        if t == 2:
            pltpu.sync_copy(out_v, emb_h.at[s * FPS + k])

    bbase = s * BSEG

    @pl.when(s < 15)
    def _bd0():
        pltpu.sync_copy(
            bias_h.at[0, pl.ds(pl.multiple_of(bbase, 128), BSEG)],
            buf_a.at[pl.ds(0, BSEG)])

    @pl.when(s == 15)
    def _bd1():
        pltpu.sync_copy(bias_h.at[0, pl.ds(15 * BSEG, V - 32 - 15 * BSEG)],
                        buf_a.at[pl.ds(0, V - 32 - 15 * BSEG)])
        pltpu.sync_copy(tail_h.at[pl.ds(D * 128, 128)],
                        buf_a.at[pl.ds(V - 32 - 15 * BSEG, 128)])

    @plsc.parallel_loop(0, NG, step=1, unroll=8)
    def bias_body(i):
        iv = ids_v[pl.ds(i * 16, 16)]
        m = (iv >= bbase) & (iv < bbase + BSEG)
        g = plsc.load_gather(buf_a, [iv - bbase], mask=m)
        out_v[pl.ds(i * 16, 16)] = jnp.where(m, g, 0.0)

    pltpu.sync_copy(out_v, bias_out_h.at[s])


def _sc_gather(uid, iid, U1t, Q1t, A1t, B1t, Ut_tail, Qt_tail):
    mesh = plsc.VectorSubcoreMesh(core_axis_name="c", subcore_axis_name="s")

    @functools.partial(
        pl.kernel,
        out_type=[
            jax.ShapeDtypeStruct((D, BATCH), jnp.float32),
            jax.ShapeDtypeStruct((D, BATCH), jnp.float32),
            jax.ShapeDtypeStruct((16, BATCH), jnp.float32),
            jax.ShapeDtypeStruct((16, BATCH), jnp.float32),
        ],
        mesh=mesh,
        scratch_types=[
            pltpu.VMEM((BATCH,), jnp.int32),
            pltpu.VMEM((SEG + 128,), jnp.float32),
            pltpu.VMEM((SEG + 128,), jnp.float32),
            pltpu.VMEM((BATCH,), jnp.float32),
            pltpu.SemaphoreType.DMA,
            pltpu.SemaphoreType.DMA,
        ],
        compiler_params=pltpu.CompilerParams(needs_layout_passes=False),
    )
    def k(uid_h, iid_h, u_h, q_h, a_h, b_h, ut_h, qt_h,
          ueT_h, ieT_h, ubp_h, ibp_h,
          ids_v, buf_a, buf_b, out_v, sem_a, sem_b):
        s = lax.axis_index("s")
        c = lax.axis_index("c")

        @pl.when(c == 0)
        def _user():
            _do_table(u_h, ut_h, a_h, uid_h, ueT_h, ubp_h, s,
                      ids_v, buf_a, buf_b, out_v, sem_a, sem_b)

        @pl.when(c == 1)
        def _item():
            _do_table(q_h, qt_h, b_h, iid_h, ieT_h, ibp_h, s,
                      ids_v, buf_a, buf_b, out_v, sem_a, sem_b)

    return k(uid, iid, U1t, Q1t, A1t, B1t, Ut_tail, Qt_tail)


def _tc_body(uT_ref, qT_ref, ubp_ref, ibp_ref, w1_ref, b1_ref, w2_ref,
             b2_ref, pred_ref, score_ref):
    uT = uT_ref[...]
    qT = qT_ref[...]
    pT = uT * qT
    pred_ref[...] = (jnp.sum(pT, axis=0) + jnp.sum(ubp_ref[...], axis=0)
                     + jnp.sum(ibp_ref[...], axis=0))
    xT = jnp.concatenate([uT, qT, pT], axis=0)
    h = lax.dot_general(w1_ref[...], xT, (((1,), (0,)), ((), ())),
                        preferred_element_type=jnp.float32)
    h = jnp.maximum(h + b1_ref[...][:, None], 0.0)
    score_ref[...] = jnp.sum(h * w2_ref[...][:, None], axis=0) + b2_ref[0]


def _tc_dense(ueT, ieT, ubp, ibp, W1, b1, W2r, b2):
    bb = 2048
    grid = (BATCH // bb,)
    return pl.pallas_call(
        _tc_body,
        grid=grid,
        in_specs=[
            pl.BlockSpec((D, bb), lambda i: (0, i)),
            pl.BlockSpec((D, bb), lambda i: (0, i)),
            pl.BlockSpec((16, bb), lambda i: (0, i)),
            pl.BlockSpec((16, bb), lambda i: (0, i)),
            pl.BlockSpec((H1, 3 * D), lambda i: (0, 0)),
            pl.BlockSpec((H1,), lambda i: (0,)),
            pl.BlockSpec((H1,), lambda i: (0,)),
            pl.BlockSpec(memory_space=pltpu.SMEM),
        ],
        out_specs=[
            pl.BlockSpec((bb,), lambda i: (i,)),
            pl.BlockSpec((bb,), lambda i: (i,)),
        ],
        out_shape=[
            jax.ShapeDtypeStruct((BATCH,), jnp.float32),
            jax.ShapeDtypeStruct((BATCH,), jnp.float32),
        ],
    )(ueT, ieT, ubp, ibp, W1, b1, W2r, b2)


def kernel(user_ids, item_ids, U1, Q1, A1, B1, W1, b1, W2, b2):
    uid = user_ids.astype(jnp.int32)
    iid = item_ids.astype(jnp.int32)
    ut_tail = jnp.pad(
        jnp.concatenate([U1[V - 32:].T, A1[V - 32:].reshape(1, 32)], 0),
        ((0, 0), (0, 96))).reshape(-1)
    qt_tail = jnp.pad(
        jnp.concatenate([Q1[V - 32:].T, B1[V - 32:].reshape(1, 32)], 0),
        ((0, 0), (0, 96))).reshape(-1)
    ueT, ieT, ubp, ibp = _sc_gather(uid, iid, U1.T, Q1.T, A1.T, B1.T,
                                    ut_tail, qt_tail)
    pred, score = _tc_dense(ueT, ieT, ubp, ibp, W1, b1, W2.reshape(-1), b2)
    return (pred, score)


# speedup vs baseline: 1.0242x; 1.0242x over previous
import functools

import jax
import jax.numpy as jnp
from jax import lax
from jax.experimental import pallas as pl
from jax.experimental.pallas import tpu as pltpu
from jax.experimental.pallas import tpu_sc as plsc

BATCH = 16384
D = 64
H1 = 128
V = 100000
FPS = 4
OCHUNK = 8192
BSEG = 6400


def _do_table(tbl_h, tail_h, bias_h, ids_h, emb_h, bias_out_h, s,
              ids_v, slice_v, out_v):
    pltpu.sync_copy(ids_h, ids_v)
    for k in range(FPS):
        f = s * FPS + k
        pltpu.sync_copy(tbl_h.at[f], slice_v)
        for half in range(BATCH // OCHUNK):
            col0 = half * OCHUNK

            @plsc.parallel_loop(0, OCHUNK // 16, step=1, unroll=8)
            def body(i):
                iv = ids_v[pl.ds(col0 + i * 16, 16)]
                out_v[pl.ds(i * 16, 16)] = plsc.load_gather(slice_v, [iv])

            pltpu.sync_copy(out_v, emb_h.at[f, pl.ds(col0, OCHUNK)])

    bbase = s * BSEG

    @pl.when(s < 15)
    def _bd0():
        pltpu.sync_copy(
            bias_h.at[0, pl.ds(pl.multiple_of(bbase, 128), BSEG)],
            slice_v.at[pl.ds(0, BSEG)])

    @pl.when(s == 15)
    def _bd1():
        pltpu.sync_copy(bias_h.at[0, pl.ds(15 * BSEG, V - 32 - 15 * BSEG)],
                        slice_v.at[pl.ds(0, V - 32 - 15 * BSEG)])
        pltpu.sync_copy(tail_h.at[pl.ds(D * 128, 128)],
                        slice_v.at[pl.ds(V - 32 - 15 * BSEG, 128)])

    for half in range(BATCH // OCHUNK):
        col0 = half * OCHUNK

        @plsc.parallel_loop(0, OCHUNK // 16, step=1, unroll=8)
        def bias_body(i):
            iv = ids_v[pl.ds(col0 + i * 16, 16)]
            m = (iv >= bbase) & (iv < bbase + BSEG)
            g = plsc.load_gather(slice_v, [iv - bbase], mask=m)
            out_v[pl.ds(i * 16, 16)] = jnp.where(m, g, 0.0)

        pltpu.sync_copy(out_v, bias_out_h.at[s, pl.ds(col0, OCHUNK)])


def _sc_gather(uid, iid, U1t, Q1t, A1t, B1t, Ut_tail, Qt_tail):
    mesh = plsc.VectorSubcoreMesh(core_axis_name="c", subcore_axis_name="s")

    @functools.partial(
        pl.kernel,
        out_type=[
            jax.ShapeDtypeStruct((D, BATCH), jnp.float32),
            jax.ShapeDtypeStruct((D, BATCH), jnp.float32),
            jax.ShapeDtypeStruct((16, BATCH), jnp.float32),
            jax.ShapeDtypeStruct((16, BATCH), jnp.float32),
        ],
        mesh=mesh,
        scratch_types=[
            pltpu.VMEM((BATCH,), jnp.int32),
            pltpu.VMEM((V,), jnp.float32),
            pltpu.VMEM((OCHUNK,), jnp.float32),
        ],
        compiler_params=pltpu.CompilerParams(needs_layout_passes=False),
    )
    def k(uid_h, iid_h, u_h, q_h, a_h, b_h, ut_h, qt_h,
          ueT_h, ieT_h, ubp_h, ibp_h, ids_v, slice_v, out_v):
        s = lax.axis_index("s")
        c = lax.axis_index("c")

        @pl.when(c == 0)
        def _user():
            _do_table(u_h, ut_h, a_h, uid_h, ueT_h, ubp_h, s,
                      ids_v, slice_v, out_v)

        @pl.when(c == 1)
        def _item():
            _do_table(q_h, qt_h, b_h, iid_h, ieT_h, ibp_h, s,
                      ids_v, slice_v, out_v)

    return k(uid, iid, U1t, Q1t, A1t, B1t, Ut_tail, Qt_tail)


def _tc_body(uT_ref, qT_ref, ubp_ref, ibp_ref, w1_ref, b1_ref, w2_ref,
             b2_ref, pred_ref, score_ref):
    uT = uT_ref[...]
    qT = qT_ref[...]
    pT = uT * qT
    pred_ref[...] = (jnp.sum(pT, axis=0) + jnp.sum(ubp_ref[...], axis=0)
                     + jnp.sum(ibp_ref[...], axis=0))
    xT = jnp.concatenate([uT, qT, pT], axis=0)
    h = lax.dot_general(w1_ref[...], xT, (((1,), (0,)), ((), ())),
                        preferred_element_type=jnp.float32)
    h = jnp.maximum(h + b1_ref[...][:, None], 0.0)
    score_ref[...] = jnp.sum(h * w2_ref[...][:, None], axis=0) + b2_ref[0]


def _tc_dense(ueT, ieT, ubp, ibp, W1, b1, W2r, b2):
    bb = 2048
    grid = (BATCH // bb,)
    return pl.pallas_call(
        _tc_body,
        grid=grid,
        in_specs=[
            pl.BlockSpec((D, bb), lambda i: (0, i)),
            pl.BlockSpec((D, bb), lambda i: (0, i)),
            pl.BlockSpec((16, bb), lambda i: (0, i)),
            pl.BlockSpec((16, bb), lambda i: (0, i)),
            pl.BlockSpec((H1, 3 * D), lambda i: (0, 0)),
            pl.BlockSpec((H1,), lambda i: (0,)),
            pl.BlockSpec((H1,), lambda i: (0,)),
            pl.BlockSpec(memory_space=pltpu.SMEM),
        ],
        out_specs=[
            pl.BlockSpec((bb,), lambda i: (i,)),
            pl.BlockSpec((bb,), lambda i: (i,)),
        ],
        out_shape=[
            jax.ShapeDtypeStruct((BATCH,), jnp.float32),
            jax.ShapeDtypeStruct((BATCH,), jnp.float32),
        ],
    )(ueT, ieT, ubp, ibp, W1, b1, W2r, b2)


def kernel(user_ids, item_ids, U1, Q1, A1, B1, W1, b1, W2, b2):
    uid = user_ids.astype(jnp.int32)
    iid = item_ids.astype(jnp.int32)
    ut_tail = jnp.pad(
        jnp.concatenate([U1[V - 32:].T, A1[V - 32:].reshape(1, 32)], 0),
        ((0, 0), (0, 96))).reshape(-1)
    qt_tail = jnp.pad(
        jnp.concatenate([Q1[V - 32:].T, B1[V - 32:].reshape(1, 32)], 0),
        ((0, 0), (0, 96))).reshape(-1)
    ueT, ieT, ubp, ibp = _sc_gather(uid, iid, U1.T, Q1.T, A1.T, B1.T,
                                    ut_tail, qt_tail)
    pred, score = _tc_dense(ueT, ieT, ubp, ibp, W1, b1, W2.reshape(-1), b2)
    return (pred, score)
